# baseline (speedup 1.0000x reference)
import jax
import jax.numpy as jnp
from jax import lax
from jax.experimental import pallas as pl
from jax.experimental.pallas import tpu as pltpu
from jax.experimental.pallas import tpu_sc as plsc

_N = 10000
_D = 128
_E = 320000
_NSUB = 16
_NCORE = 2
_EPW = _E // (_NCORE * _NSUB)
_RPS = _N // _NSUB


def _make_agg_resident(width, chunk, halves=1, nbuf=2):
    nchunk = _EPW // chunk
    nres = nchunk // halves
    step = 2 * nbuf if nbuf % 2 else nbuf
    assert nchunk * chunk == _EPW and nres * halves == nchunk
    mesh = plsc.VectorSubcoreMesh(core_axis_name="c", subcore_axis_name="s")

    def body(ei, table, zeros, out, acc, sidx, didx, rows, *gsems):
        c = lax.axis_index("c")
        s = lax.axis_index("s")
        w = c * _NSUB + s
        base = s * _RPS
        pltpu.async_copy(ei.at[0, w, pl.ds(0, nres)], sidx, gsems[0])
        pltpu.async_copy(ei.at[1, w, pl.ds(0, nres)], didx, gsems[1])
        pltpu.sync_copy(zeros.at[pl.ds(base, _RPS)], acc.at[pl.ds(base, _RPS)])
        pltpu.make_async_copy(ei.at[0, 0, pl.ds(0, nres)], sidx,
                              gsems[0]).wait()
        pltpu.make_async_copy(ei.at[1, 0, pl.ds(0, nres)], didx,
                              gsems[1]).wait()
        for b in range(nbuf):
            pltpu.async_copy(table.at[sidx.at[b]], rows.at[b], gsems[b])
        plsc.subcore_barrier()

        for h in range(halves):
            if h > 0:
                pltpu.sync_copy(ei.at[0, w, pl.ds(h * nres, nres)], sidx)
                pltpu.sync_copy(ei.at[1, w, pl.ds(h * nres, nres)], didx)
                for b in range(nbuf):
                    pltpu.async_copy(table.at[sidx.at[b]], rows.at[b],
                                     gsems[b])

            @pl.loop(0, nres, step=step)
            def _(j):
                for k in range(step):
                    b = k % nbuf

                    @pl.when(j + k < nres)
                    def _(k=k, b=b):
                        pltpu.make_async_copy(
                            table.at[pl.ds(0, chunk)], rows.at[b],
                            gsems[b]).wait()
                        pltpu.sync_copy(rows.at[b], acc.at[didx.at[j + k]],
                                        add=True)

                        @pl.when(j + k + nbuf < nres)
                        def _():
                            pltpu.async_copy(
                                table.at[sidx.at[j + k + nbuf]], rows.at[b],
                                gsems[b])

        plsc.subcore_barrier()
        pltpu.sync_copy(acc.at[pl.ds(base, _RPS)],
                        out.at[c, pl.ds(base, _RPS)])

    return pl.kernel(
        body,
        out_type=jax.ShapeDtypeStruct((_NCORE, _N, width), jnp.float32),
        mesh=mesh,
        compiler_params=pltpu.CompilerParams(use_tc_tiling_on_sc=False),
        scratch_types=[
            pltpu.VMEM_SHARED((_N, width), jnp.float32),
            pltpu.VMEM((nres, chunk), jnp.int32),
            pltpu.VMEM((nres, chunk), jnp.int32),
            pltpu.VMEM((nbuf, chunk, width), jnp.float32),
        ] + [pltpu.SemaphoreType.DMA] * max(nbuf, 2),
    )


_CHUNK = 100
_agg_l1 = _make_agg_resident(_D + 16, 80, halves=5, nbuf=3)
_agg = _make_agg_resident(_D, 80, halves=5, nbuf=4)


def _layer1_body(p_ref, x_ref, wl_ref, bl_ref, wr_ref, h_ref, inv_ref):
    msum = p_ref[0, :, :_D] + p_ref[1, :, :_D]
    cnt = p_ref[0, :, _D:_D + 1] + p_ref[1, :, _D:_D + 1]
    inv = 1.0 / jnp.maximum(cnt, 1.0)
    mean = msum * inv
    h = jnp.dot(mean, wl_ref[...], preferred_element_type=jnp.float32)
    h = h + bl_ref[...] + jnp.dot(x_ref[...], wr_ref[...],
                                  preferred_element_type=jnp.float32)
    h_ref[...] = jnp.maximum(h, 0.0)
    inv_ref[...] = inv


_layer1_tc = pl.pallas_call(
    _layer1_body,
    out_shape=(
        jax.ShapeDtypeStruct((_N, _D), jnp.float32),
        jax.ShapeDtypeStruct((_N, 1), jnp.float32),
    ),
)


def _make_layer23(relu):
    def body(p_ref, inv_ref, x_ref, wl_ref, bl_ref, wr_ref, o_ref):
        mean = (p_ref[0] + p_ref[1]) * inv_ref[...]
        h = jnp.dot(mean, wl_ref[...], preferred_element_type=jnp.float32)
        h = h + bl_ref[...] + jnp.dot(x_ref[...], wr_ref[...],
                                      preferred_element_type=jnp.float32)
        o_ref[...] = jnp.maximum(h, 0.0) if relu else h

    return pl.pallas_call(
        body, out_shape=jax.ShapeDtypeStruct((_N, _D), jnp.float32))


_layer2_tc = _make_layer23(True)
_layer3_tc = _make_layer23(False)


def kernel(x, edge_index, Wl1, bl1, Wr1, Wl2, bl2, Wr2, Wl3, bl3, Wr3):
    ei = edge_index.astype(jnp.int32)
    eir = ei.reshape(2, _NCORE * _NSUB, _EPW // _CHUNK, _CHUNK)
    ei_l1 = ei.reshape(2, _NCORE * _NSUB, _EPW // 80, 80)
    xa = jnp.concatenate(
        [x, jnp.ones((_N, 1), jnp.float32), jnp.zeros((_N, 15), jnp.float32)],
        axis=1)
    z_l1 = jnp.zeros((_N, _D + 16), jnp.float32)
    z = jnp.zeros((_N, _D), jnp.float32)
    bl1r = bl1.reshape(1, _D)
    bl2r = bl2.reshape(1, _D)
    bl3r = bl3.reshape(1, _D)

    p1 = _agg_l1(ei_l1, xa, z_l1)
    h1, inv = _layer1_tc(p1, x, Wl1, bl1r, Wr1)
    p2 = _agg(ei_l1, h1, z)
    h2 = _layer2_tc(p2, inv, h1, Wl2, bl2r, Wr2)
    p3 = _agg(ei_l1, h2, z)
    return _layer3_tc(p3, inv, h2, Wl3, bl3r, Wr3)


# speedup vs baseline: 1.0343x; 1.0343x over previous
import jax
import jax.numpy as jnp
from jax import lax
from jax.experimental import pallas as pl
from jax.experimental.pallas import tpu as pltpu
from jax.experimental.pallas import tpu_sc as plsc

_N = 10000
_D = 128
_E = 320000
_NSUB = 16
_NCORE = 2
_EPW = _E // (_NCORE * _NSUB)
_RPS = _N // _NSUB


def _make_agg_resident(width, chunk, halves=1, nbuf=2):
    nchunk = _EPW // chunk
    nres = nchunk // halves
    step = 2 * nbuf if nbuf % 2 else nbuf
    assert nchunk * chunk == _EPW and nres * halves == nchunk
    mesh = plsc.VectorSubcoreMesh(core_axis_name="c", subcore_axis_name="s")

    def body(ei, table, zeros, out, acc, sidx, didx, rows, *gsems):
        c = lax.axis_index("c")
        s = lax.axis_index("s")
        w = c * _NSUB + s
        base = s * _RPS
        pltpu.async_copy(ei.at[0, w, pl.ds(0, nres)], sidx, gsems[0])
        pltpu.async_copy(ei.at[1, w, pl.ds(0, nres)], didx, gsems[1])
        pltpu.sync_copy(zeros.at[pl.ds(base, _RPS)], acc.at[pl.ds(base, _RPS)])
        pltpu.make_async_copy(ei.at[0, 0, pl.ds(0, nres)], sidx,
                              gsems[0]).wait()
        pltpu.make_async_copy(ei.at[1, 0, pl.ds(0, nres)], didx,
                              gsems[1]).wait()
        for b in range(nbuf):
            pltpu.async_copy(table.at[sidx.at[b]], rows.at[b], gsems[b])
        plsc.subcore_barrier()

        for h in range(halves):
            if h > 0:
                pltpu.sync_copy(ei.at[0, w, pl.ds(h * nres, nres)], sidx)
                pltpu.sync_copy(ei.at[1, w, pl.ds(h * nres, nres)], didx)
                for b in range(nbuf):
                    pltpu.async_copy(table.at[sidx.at[b]], rows.at[b],
                                     gsems[b])

            @pl.loop(0, nres, step=step)
            def _(j):
                for k in range(step):
                    b = k % nbuf

                    @pl.when(j + k < nres)
                    def _(k=k, b=b):
                        pltpu.make_async_copy(
                            table.at[pl.ds(0, chunk)], rows.at[b],
                            gsems[b]).wait()
                        pltpu.sync_copy(rows.at[b], acc.at[didx.at[j + k]],
                                        add=True)

                        @pl.when(j + k + nbuf < nres)
                        def _():
                            pltpu.async_copy(
                                table.at[sidx.at[j + k + nbuf]], rows.at[b],
                                gsems[b])

        plsc.subcore_barrier()
        pltpu.sync_copy(acc.at[pl.ds(base, _RPS)],
                        out.at[c, pl.ds(base, _RPS)])

    return pl.kernel(
        body,
        out_type=jax.ShapeDtypeStruct((_NCORE, _N, width), jnp.float32),
        mesh=mesh,
        compiler_params=pltpu.CompilerParams(use_tc_tiling_on_sc=False),
        scratch_types=[
            pltpu.VMEM_SHARED((_N, width), jnp.float32),
            pltpu.VMEM((nres, chunk), jnp.int32),
            pltpu.VMEM((nres, chunk), jnp.int32),
            pltpu.VMEM((nbuf, chunk, width), jnp.float32),
        ] + [pltpu.SemaphoreType.DMA] * max(nbuf, 2),
    )


_CHUNK = 100
_agg_l1 = _make_agg_resident(_D + 16, 80, halves=5, nbuf=3)
_agg = _make_agg_resident(_D, _CHUNK, halves=2, nbuf=3)


def _resid_body(x_ref, wr_ref, bl_ref, r_ref):
    r_ref[...] = jnp.dot(x_ref[...], wr_ref[...],
                         preferred_element_type=jnp.float32) + bl_ref[...]


_resid_tc = pl.pallas_call(
    _resid_body, out_shape=jax.ShapeDtypeStruct((_N, _D), jnp.float32))


def _layer1_body(p_ref, r_ref, wl_ref, h_ref, inv_ref):
    msum = p_ref[0, :, :_D] + p_ref[1, :, :_D]
    cnt = p_ref[0, :, _D:_D + 1] + p_ref[1, :, _D:_D + 1]
    inv = 1.0 / jnp.maximum(cnt, 1.0)
    mean = msum * inv
    h = jnp.dot(mean, wl_ref[...], preferred_element_type=jnp.float32)
    h_ref[...] = jnp.maximum(h + r_ref[...], 0.0)
    inv_ref[...] = inv


_layer1_tc = pl.pallas_call(
    _layer1_body,
    out_shape=(
        jax.ShapeDtypeStruct((_N, _D), jnp.float32),
        jax.ShapeDtypeStruct((_N, 1), jnp.float32),
    ),
)


def _make_layer23(relu):
    def body(p_ref, inv_ref, r_ref, wl_ref, o_ref):
        mean = (p_ref[0] + p_ref[1]) * inv_ref[...]
        h = jnp.dot(mean, wl_ref[...],
                    preferred_element_type=jnp.float32) + r_ref[...]
        o_ref[...] = jnp.maximum(h, 0.0) if relu else h

    return pl.pallas_call(
        body, out_shape=jax.ShapeDtypeStruct((_N, _D), jnp.float32))


_layer2_tc = _make_layer23(True)
_layer3_tc = _make_layer23(False)


def kernel(x, edge_index, Wl1, bl1, Wr1, Wl2, bl2, Wr2, Wl3, bl3, Wr3):
    ei = edge_index.astype(jnp.int32)
    eir = ei.reshape(2, _NCORE * _NSUB, _EPW // _CHUNK, _CHUNK)
    ei_l1 = ei.reshape(2, _NCORE * _NSUB, _EPW // 80, 80)
    xa = jnp.concatenate(
        [x, jnp.ones((_N, 1), jnp.float32), jnp.zeros((_N, 15), jnp.float32)],
        axis=1)
    z_l1 = jnp.zeros((_N, _D + 16), jnp.float32)
    z = jnp.zeros((_N, _D), jnp.float32)
    bl1r = bl1.reshape(1, _D)
    bl2r = bl2.reshape(1, _D)
    bl3r = bl3.reshape(1, _D)

    r1 = _resid_tc(x, Wr1, bl1r)
    p1 = _agg_l1(ei_l1, xa, z_l1)
    h1, inv = _layer1_tc(p1, r1, Wl1)
    r2 = _resid_tc(h1, Wr2, bl2r)
    p2 = _agg(eir, h1, z)
    h2 = _layer2_tc(p2, inv, r2, Wl2)
    r3 = _resid_tc(h2, Wr3, bl3r)
    p3 = _agg(eir, h2, z)
    return _layer3_tc(p3, inv, r3, Wl3)


# speedup vs baseline: 1.0457x; 1.0109x over previous
import jax
import jax.numpy as jnp
from jax import lax
from jax.experimental import pallas as pl
from jax.experimental.pallas import tpu as pltpu
from jax.experimental.pallas import tpu_sc as plsc

_N = 10000
_D = 128
_E = 320000
_NSUB = 16
_NCORE = 2
_EPW = _E // (_NCORE * _NSUB)
_RPS = _N // _NSUB


def _make_agg_resident(width, chunk, halves=1, nbuf=2):
    nchunk = _EPW // chunk
    nres = nchunk // halves
    step = 2 * nbuf if nbuf % 2 else nbuf
    assert nchunk * chunk == _EPW and nres * halves == nchunk
    mesh = plsc.VectorSubcoreMesh(core_axis_name="c", subcore_axis_name="s")

    def body(ei, table, zeros, out, acc, sidx, didx, rows, *gsems):
        c = lax.axis_index("c")
        s = lax.axis_index("s")
        w = c * _NSUB + s
        base = s * _RPS
        pltpu.async_copy(ei.at[0, w, pl.ds(0, nres)], sidx, gsems[0])
        pltpu.async_copy(ei.at[1, w, pl.ds(0, nres)], didx, gsems[1])
        pltpu.sync_copy(zeros.at[pl.ds(base, _RPS)], acc.at[pl.ds(base, _RPS)])
        pltpu.make_async_copy(ei.at[0, 0, pl.ds(0, nres)], sidx,
                              gsems[0]).wait()
        pltpu.make_async_copy(ei.at[1, 0, pl.ds(0, nres)], didx,
                              gsems[1]).wait()
        for b in range(nbuf):
            pltpu.async_copy(table.at[sidx.at[b]], rows.at[b], gsems[b])
        plsc.subcore_barrier()

        for h in range(halves):
            if h > 0:
                pltpu.sync_copy(ei.at[0, w, pl.ds(h * nres, nres)], sidx)
                pltpu.sync_copy(ei.at[1, w, pl.ds(h * nres, nres)], didx)
                for b in range(nbuf):
                    pltpu.async_copy(table.at[sidx.at[b]], rows.at[b],
                                     gsems[b])

            @pl.loop(0, nres, step=step)
            def _(j):
                for k in range(step):
                    b = k % nbuf

                    @pl.when(j + k < nres)
                    def _(k=k, b=b):
                        pltpu.make_async_copy(
                            table.at[pl.ds(0, chunk)], rows.at[b],
                            gsems[b]).wait()
                        pltpu.sync_copy(rows.at[b], acc.at[didx.at[j + k]],
                                        add=True)

                        @pl.when(j + k + nbuf < nres)
                        def _():
                            pltpu.async_copy(
                                table.at[sidx.at[j + k + nbuf]], rows.at[b],
                                gsems[b])

        plsc.subcore_barrier()
        pltpu.sync_copy(acc.at[pl.ds(base, _RPS)],
                        out.at[c, pl.ds(base, _RPS)])

    return pl.kernel(
        body,
        out_type=jax.ShapeDtypeStruct((_NCORE, _N, width), jnp.float32),
        mesh=mesh,
        compiler_params=pltpu.CompilerParams(use_tc_tiling_on_sc=False),
        scratch_types=[
            pltpu.VMEM_SHARED((_N, width), jnp.float32),
            pltpu.VMEM((nres, chunk), jnp.int32),
            pltpu.VMEM((nres, chunk), jnp.int32),
            pltpu.VMEM((nbuf, chunk, width), jnp.float32),
        ] + [pltpu.SemaphoreType.DMA] * max(nbuf, 2),
    )


_CHUNK = 100
_agg_l1 = _make_agg_resident(_D + 16, 80, halves=5, nbuf=3)
_agg = _make_agg_resident(_D, _CHUNK, halves=2, nbuf=3)


def _layer1_body(p_ref, x_ref, wl_ref, bl_ref, wr_ref, h_ref, inv_ref):
    msum = p_ref[0, :, :_D] + p_ref[1, :, :_D]
    cnt = p_ref[0, :, _D:_D + 1] + p_ref[1, :, _D:_D + 1]
    inv = 1.0 / jnp.maximum(cnt, 1.0)
    mean = msum * inv
    h = jnp.dot(mean, wl_ref[...], preferred_element_type=jnp.float32)
    h = h + bl_ref[...] + jnp.dot(x_ref[...], wr_ref[...],
                                  preferred_element_type=jnp.float32)
    h_ref[...] = jnp.maximum(h, 0.0)
    inv_ref[...] = inv


_layer1_tc = pl.pallas_call(
    _layer1_body,
    out_shape=(
        jax.ShapeDtypeStruct((_N, _D), jnp.float32),
        jax.ShapeDtypeStruct((_N, 1), jnp.float32),
    ),
)


def _make_layer23(relu):
    def body(p_ref, inv_ref, x_ref, wl_ref, bl_ref, wr_ref, o_ref):
        mean = (p_ref[0] + p_ref[1]) * inv_ref[...]
        h = jnp.dot(mean, wl_ref[...], preferred_element_type=jnp.float32)
        h = h + bl_ref[...] + jnp.dot(x_ref[...], wr_ref[...],
                                      preferred_element_type=jnp.float32)
        o_ref[...] = jnp.maximum(h, 0.0) if relu else h

    return pl.pallas_call(
        body, out_shape=jax.ShapeDtypeStruct((_N, _D), jnp.float32))


_layer2_tc = _make_layer23(True)
_layer3_tc = _make_layer23(False)


def kernel(x, edge_index, Wl1, bl1, Wr1, Wl2, bl2, Wr2, Wl3, bl3, Wr3):
    ei = edge_index.astype(jnp.int32)
    eir = ei.reshape(2, _NCORE * _NSUB, _EPW // _CHUNK, _CHUNK)
    ei_l1 = ei.reshape(2, _NCORE * _NSUB, _EPW // 80, 80)
    xa = jnp.concatenate(
        [x, jnp.ones((_N, 1), jnp.float32), jnp.zeros((_N, 15), jnp.float32)],
        axis=1)
    z_l1 = jnp.zeros((_N, _D + 16), jnp.float32)
    z = jnp.zeros((_N, _D), jnp.float32)
    bl1r = bl1.reshape(1, _D)
    bl2r = bl2.reshape(1, _D)
    bl3r = bl3.reshape(1, _D)

    p1 = _agg_l1(ei_l1, xa, z_l1)
    h1, inv = _layer1_tc(p1, x, Wl1, bl1r, Wr1)
    p2 = _agg(eir, h1, z)
    h2 = _layer2_tc(p2, inv, h1, Wl2, bl2r, Wr2)
    p3 = _agg(eir, h2, z)
    return _layer3_tc(p3, inv, h2, Wl3, bl3r, Wr3)
